# Initial kernel scaffold; baseline (speedup 1.0000x reference)
import jax
import jax.numpy as jnp
from jax.experimental import pallas as pl


def kernel(input_ids, position_ids, token_table, position_table):
    raise NotImplementedError("write your pallas kernel here")


# speedup vs baseline: 1.2782x; 1.2782x over previous
import functools

import jax
import jax.numpy as jnp
from jax import lax
from jax.experimental import pallas as pl
from jax.experimental.pallas import tpu as pltpu
from jax.experimental.pallas import tpu_sc as plsc

B = 1024
L = 77
VOCAB = 49408
MAXLEN = 77
D = 1024

BL = B * L
NC = 2
NS = 16
NW = NC * NS
PER_W = BL // NW
C = 56
NCHUNK = PER_W // C
DL = D // 16


def _sc_body(tok_ids, pos_ids, tok_tab, pos_tab, out,
             tix, pix, trows, prows, sem):
    wid = lax.axis_index("s") * NC + lax.axis_index("c")
    base = wid * PER_W
    pltpu.sync_copy(tok_ids.at[pl.ds(base, PER_W)], tix)
    pltpu.sync_copy(pos_ids.at[pl.ds(base, PER_W)], pix)

    def chunk(g, carry):
        off = g * C
        pltpu.async_copy(tok_tab.at[tix.at[pl.ds(off, C)]], trows, sem).wait()
        pltpu.async_copy(pos_tab.at[pix.at[pl.ds(off, C)]], prows, sem).wait()

        def row(r, c2):
            for j in range(DL):
                s = pl.ds(j * 16, 16)
                trows[r, s] = trows[r, s] + prows[r, s]
            return c2

        lax.fori_loop(0, C, row, 0)
        pltpu.sync_copy(trows, out.at[pl.ds(base + off, C)])
        return carry

    lax.fori_loop(0, NCHUNK, chunk, 0)


@jax.jit
def _embed(tok_ids_flat, pos_ids_flat, token_table, position_table):
    mesh = plsc.VectorSubcoreMesh(core_axis_name="c", subcore_axis_name="s")
    k = functools.partial(
        pl.kernel,
        mesh=mesh,
        out_type=jax.ShapeDtypeStruct((BL, D), jnp.float32),
        scratch_types=[
            pltpu.VMEM((PER_W,), jnp.int32),
            pltpu.VMEM((PER_W,), jnp.int32),
            pltpu.VMEM((C, D), jnp.float32),
            pltpu.VMEM((C, D), jnp.float32),
            pltpu.SemaphoreType.DMA,
        ],
    )(_sc_body)
    return k(tok_ids_flat, pos_ids_flat, token_table, position_table)


def kernel(input_ids, position_ids, token_table, position_table):
    tok_flat = jnp.reshape(input_ids.astype(jnp.int32), (BL,))
    pos_flat = jnp.reshape(position_ids.astype(jnp.int32), (BL,))
    out = _embed(tok_flat, pos_flat, token_table, position_table)
    return jnp.reshape(out, (B, L, D))
